# baseline (speedup 1.0000x reference)
import functools

import jax
import jax.numpy as jnp
from jax import lax
from jax.experimental import pallas as pl
from jax.experimental.pallas import tpu as pltpu
from jax.experimental.pallas import tpu_sc as plsc

VOCAB = 1000000
CONTEXT = 200
EMBED = 64
BATCH = 4096
SEQ = 200

B = BATCH * SEQ
NC, NS = 2, 16
NW = NC * NS
RPW = B // NW
SUPER = 400
NSUP = RPW // SUPER
NVG = SUPER // 16
NBUF = 3


def _sc_body(idx_hbm, tok_hbm, pos_hbm, out_hbm,
             idx_v, pos_v, gbuf, g0, g1, g2, s0, s1, s2):
    gsems = (g0, g1, g2)
    ssems = (s0, s1, s2)
    wid = lax.axis_index("s") * NC + lax.axis_index("c")

    pltpu.sync_copy(idx_hbm.at[wid], idx_v)
    pltpu.sync_copy(pos_hbm, pos_v)

    def fire_gathers(s, b):
        base = s * SUPER
        for q in range(NVG):
            vals = idx_v[pl.ds(base + q * 16, 16)]
            pltpu.async_copy(
                tok_hbm.at[vals], gbuf.at[b, pl.ds(q * 16, 16)], gsems[b]
            )

    def fire_store(s, b):
        row0 = wid * RPW + s * SUPER
        pltpu.async_copy(gbuf.at[b], out_hbm.at[pl.ds(row0, SUPER)], ssems[b])

    def drain(sem, b):
        pltpu.make_async_copy(
            out_hbm.at[pl.ds(0, SUPER)], gbuf.at[b], sem
        ).wait()

    def add_pos(b):
        for half in range(2):
            base = half * SEQ

            def body(r4, _):
                for q in range(4):
                    r = r4 * 4 + q
                    for k in range(EMBED // 16):
                        sl = pl.ds(k * 16, 16)
                        plsc.addupdate(gbuf.at[b, base + r, sl], pos_v[r, sl])
                return 0

            lax.fori_loop(0, SEQ // 4, body, 0)

    def process(s, b):
        drain(gsems[b], b)
        add_pos(b)
        fire_store(s, b)

        @pl.when(s + 2 < NSUP)
        def _():
            b2 = (b + 2) % NBUF

            @pl.when(s + 2 >= NBUF)
            def _():
                drain(ssems[b2], b2)

            fire_gathers(s + 2, b2)

    fire_gathers(0, 0)
    fire_gathers(1, 1)

    def step(i, carry):
        for b3 in range(NBUF):
            process(NBUF * i + b3, b3)
        return carry

    lax.fori_loop(0, (NSUP - 1) // NBUF, step, 0)

    process(NSUP - 1, (NSUP - 1) % NBUF)
    for s in range(NSUP - 3, NSUP):
        drain(ssems[s % NBUF], s % NBUF)


@jax.jit
def _tok_pos_embed(idx2, token_table, position_table):
    mesh = plsc.VectorSubcoreMesh(core_axis_name="c", subcore_axis_name="s")
    f = functools.partial(
        pl.kernel,
        out_type=jax.ShapeDtypeStruct((B, EMBED), jnp.float32),
        mesh=mesh,
        compiler_params=pltpu.CompilerParams(use_tc_tiling_on_sc=False),
        scratch_types=[
            pltpu.VMEM((RPW,), jnp.int32),
            pltpu.VMEM((CONTEXT, EMBED), jnp.float32),
            pltpu.VMEM((NBUF, SUPER, EMBED), jnp.float32),
        ] + [pltpu.SemaphoreType.DMA] * (2 * NBUF),
    )(_sc_body)
    return f(idx2, token_table, position_table)


def kernel(inputs, token_table, position_table):
    idx2 = inputs.astype(jnp.int32).reshape(NW, RPW)
    out = _tok_pos_embed(idx2, token_table, position_table)
    return out.reshape(BATCH, SEQ, EMBED)


# speedup vs baseline: 1.1285x; 1.1285x over previous
import functools

import jax
import jax.numpy as jnp
from jax import lax
from jax.experimental import pallas as pl
from jax.experimental.pallas import tpu as pltpu
from jax.experimental.pallas import tpu_sc as plsc

VOCAB = 1000000
CONTEXT = 200
EMBED = 64
BATCH = 4096
SEQ = 200

B = BATCH * SEQ
NC, NS = 2, 16
NW = NC * NS
RPW = B // NW
SUPER = 400
NSUP = RPW // SUPER
NVG = SUPER // 16
NBUF = 3


def _sc_body(idx_hbm, tok_hbm, pos_hbm, out_hbm,
             idx_v, pos_v, gbuf, g0, g1, g2, s0, s1, s2):
    wid = lax.axis_index("s") * NC + lax.axis_index("c")
    pltpu.sync_copy(idx_hbm.at[wid], idx_v)
    pltpu.sync_copy(pos_hbm, pos_v)
    row0 = wid * RPW
    pltpu.sync_copy(gbuf.at[0], out_hbm.at[pl.ds(row0, SUPER)])


@jax.jit
def _tok_pos_embed(idx2, token_table, position_table):
    mesh = plsc.VectorSubcoreMesh(core_axis_name="c", subcore_axis_name="s")
    f = functools.partial(
        pl.kernel,
        out_type=jax.ShapeDtypeStruct((B, EMBED), jnp.float32),
        mesh=mesh,
        compiler_params=pltpu.CompilerParams(use_tc_tiling_on_sc=False),
        scratch_types=[
            pltpu.VMEM((RPW,), jnp.int32),
            pltpu.VMEM((CONTEXT, EMBED), jnp.float32),
            pltpu.VMEM((NBUF, SUPER, EMBED), jnp.float32),
        ] + [pltpu.SemaphoreType.DMA] * (2 * NBUF),
    )(_sc_body)
    return f(idx2, token_table, position_table)


def kernel(inputs, token_table, position_table):
    idx2 = inputs.astype(jnp.int32).reshape(NW, RPW)
    out = _tok_pos_embed(idx2, token_table, position_table)
    return out.reshape(BATCH, SEQ, EMBED)


# speedup vs baseline: 1.3317x; 1.1800x over previous
import functools

import jax
import jax.numpy as jnp
from jax import lax
from jax.experimental import pallas as pl
from jax.experimental.pallas import tpu as pltpu
from jax.experimental.pallas import tpu_sc as plsc

VOCAB = 1000000
CONTEXT = 200
EMBED = 64
BATCH = 4096
SEQ = 200

B = BATCH * SEQ
NC, NS = 2, 16
NW = NC * NS
RPW = B // NW
SUPER = 400
NSUP = RPW // SUPER
NVG = SUPER // 16
NBUF = 3


def _sc_body(idx_hbm, tok_hbm, pos_hbm, out_hbm,
             idx_v, pos_v, gbuf, g0, g1, g2, s0, s1, s2):
    gsems = (g0, g1, g2)
    ssems = (s0, s1, s2)
    wid = lax.axis_index("s") * NC + lax.axis_index("c")

    pltpu.sync_copy(idx_hbm.at[wid], idx_v)
    pltpu.sync_copy(pos_hbm, pos_v)

    def fire_gathers(s, b):
        base = s * SUPER
        for q in range(NVG):
            vals = idx_v[pl.ds(base + q * 16, 16)]
            pltpu.async_copy(
                tok_hbm.at[vals], gbuf.at[b, pl.ds(q * 16, 16)], gsems[b]
            )

    def fire_store(s, b):
        row0 = wid * RPW + s * SUPER
        pltpu.async_copy(
            gbuf.at[b],
            out_hbm.at[pl.ds(row0, SUPER), pl.ds(0, EMBED)],
            ssems[b],
        )

    def drain(sem, b):
        pltpu.make_async_copy(
            out_hbm.at[pl.ds(0, SUPER), pl.ds(0, EMBED)], gbuf.at[b], sem
        ).wait()

    def add_pos(b):
        for half in range(2):
            base = half * SEQ

            def body(r4, _):
                for q in range(4):
                    r = r4 * 4 + q
                    for k in range(EMBED // 16):
                        sl = pl.ds(k * 16, 16)
                        plsc.addupdate(gbuf.at[b, base + r, sl], pos_v[r, sl])
                return 0

            lax.fori_loop(0, SEQ // 4, body, 0)

    def process(s, b):
        drain(gsems[b], b)
        add_pos(b)
        fire_store(s, b)

        @pl.when(s + 2 < NSUP)
        def _():
            b2 = (b + 2) % NBUF

            @pl.when(s + 2 >= NBUF)
            def _():
                drain(ssems[b2], b2)

            fire_gathers(s + 2, b2)

    fire_gathers(0, 0)
    fire_gathers(1, 1)

    def step(i, carry):
        for b3 in range(NBUF):
            process(NBUF * i + b3, b3)
        return carry

    lax.fori_loop(0, (NSUP - 1) // NBUF, step, 0)

    process(NSUP - 1, (NSUP - 1) % NBUF)
    for s in range(NSUP - 3, NSUP):
        drain(ssems[s % NBUF], s % NBUF)


@jax.jit
def _tok_pos_embed(idx2, token_table, position_table):
    mesh = plsc.VectorSubcoreMesh(core_axis_name="c", subcore_axis_name="s")
    f = functools.partial(
        pl.kernel,
        out_type=jax.ShapeDtypeStruct((B, 2 * EMBED), jnp.float32),
        mesh=mesh,
        compiler_params=pltpu.CompilerParams(use_tc_tiling_on_sc=False),
        scratch_types=[
            pltpu.VMEM((RPW,), jnp.int32),
            pltpu.VMEM((CONTEXT, EMBED), jnp.float32),
            pltpu.VMEM((NBUF, SUPER, EMBED), jnp.float32),
        ] + [pltpu.SemaphoreType.DMA] * (2 * NBUF),
    )(_sc_body)
    return f(idx2, token_table, position_table)


def kernel(inputs, token_table, position_table):
    idx2 = inputs.astype(jnp.int32).reshape(NW, RPW)
    out = _tok_pos_embed(idx2, token_table, position_table)
    return out[:, :EMBED].reshape(BATCH, SEQ, EMBED)
